# baseline (speedup 1.0000x reference)
import functools

import jax
import jax.numpy as jnp
from jax import lax
from jax.experimental import pallas as pl
from jax.experimental.pallas import tpu as pltpu

K = 1024
C = 64
HW = 1024


def _vq_kernel(xb_ref, cb_ref, codes_ref, idx_ref):
    xb = xb_ref[0].reshape(C, HW)
    cb = cb_ref[...]

    mm = lax.dot_general(cb, xb, (((1,), (0,)), ((), ())),
                         preferred_element_type=jnp.float32)
    cb_sqr = jnp.sum(cb * cb, axis=1, keepdims=True)
    x_sqr = jnp.sum(xb * xb, axis=0, keepdims=True)
    dist = cb_sqr + x_sqr - 2.0 * mm

    minval = jnp.min(dist, axis=0, keepdims=True)
    iota_f = lax.broadcasted_iota(jnp.int32, (K, HW), 0).astype(jnp.float32)
    masked = jnp.where(dist == minval, iota_f, jnp.float32(K))
    idx_f = jnp.min(masked, axis=0, keepdims=True)

    onehot = jnp.where(masked == idx_f, 1.0, 0.0)
    codes = lax.dot_general(cb, onehot, (((0,), (0,)), ((), ())),
                            preferred_element_type=jnp.float32)
    codes_ref[0] = codes.reshape(C, 32, 32)
    idx_ref[0] = idx_f.astype(jnp.int32).reshape(32, 32)


@jax.jit
def kernel(x, codebook):
    B = x.shape[0]
    codes_out, ind_out = pl.pallas_call(
        _vq_kernel,
        grid=(B,),
        in_specs=[
            pl.BlockSpec((1, C, 32, 32), lambda b: (b, 0, 0, 0)),
            pl.BlockSpec((K, C), lambda b: (0, 0)),
        ],
        out_specs=[
            pl.BlockSpec((1, C, 32, 32), lambda b: (b, 0, 0, 0)),
            pl.BlockSpec((1, 32, 32), lambda b: (b, 0, 0)),
        ],
        out_shape=[
            jax.ShapeDtypeStruct((B, C, 32, 32), jnp.float32),
            jax.ShapeDtypeStruct((B, 32, 32), jnp.int32),
        ],
    )(x, codebook)
    return codes_out, ind_out


# speedup vs baseline: 1.8433x; 1.8433x over previous
import jax
import jax.numpy as jnp
from jax import lax
from jax.experimental import pallas as pl

K = 1024
C = 64
HW = 1024


def _vq_kernel(xb_ref, cbt_ref, codes_ref, idx_ref):
    flat = xb_ref[0].reshape(HW, C)
    cbt = cbt_ref[...]

    mm = lax.dot_general(flat, cbt, (((1,), (0,)), ((), ())),
                         preferred_element_type=jnp.float32)
    x_sqr = jnp.sum(flat * flat, axis=1, keepdims=True)
    cb_sqr = jnp.sum(cbt * cbt, axis=0, keepdims=True)
    dist = (x_sqr.reshape(32, 32, 1) + cb_sqr.reshape(1, 1, K)
            - 2.0 * mm.reshape(32, 32, K))

    minval = jnp.min(dist, axis=2, keepdims=True)
    iota_f = lax.broadcasted_iota(jnp.int32, (32, 32, K), 2).astype(jnp.float32)
    masked = jnp.where(dist == minval, iota_f, jnp.float32(K))
    idx_f = jnp.min(masked, axis=2, keepdims=True)
    idx_ref[0] = idx_f.astype(jnp.int32).reshape(32, 32)

    onehot = jnp.where(masked == idx_f, 1.0, 0.0)
    codes = lax.dot_general(onehot.reshape(HW, K), cbt,
                            (((1,), (1,)), ((), ())),
                            preferred_element_type=jnp.float32)
    codes_ref[0] = codes.reshape(32, 32, C)


@jax.jit
def kernel(x, codebook):
    B = x.shape[0]
    xt = jnp.transpose(x, (0, 2, 3, 1))
    cbt = codebook.T
    codes_bhwc, ind_out = pl.pallas_call(
        _vq_kernel,
        grid=(B,),
        in_specs=[
            pl.BlockSpec((1, 32, 32, C), lambda b: (b, 0, 0, 0)),
            pl.BlockSpec((C, K), lambda b: (0, 0)),
        ],
        out_specs=[
            pl.BlockSpec((1, 32, 32, C), lambda b: (b, 0, 0, 0)),
            pl.BlockSpec((1, 32, 32), lambda b: (b, 0, 0)),
        ],
        out_shape=[
            jax.ShapeDtypeStruct((B, 32, 32, C), jnp.float32),
            jax.ShapeDtypeStruct((B, 32, 32), jnp.int32),
        ],
    )(xt, cbt)
    codes_out = jnp.transpose(codes_bhwc, (0, 3, 1, 2))
    return codes_out, ind_out


# speedup vs baseline: 1.8831x; 1.0216x over previous
import jax
import jax.numpy as jnp
from jax import lax
from jax.experimental import pallas as pl

K = 1024
C = 64
HW = 1024


def _vq_kernel(xb_ref, cbt2_ref, aug_ref, codes_ref, idx_ref):
    flat = xb_ref[0].reshape(HW, C)
    cbt2 = cbt2_ref[...]
    aug = aug_ref[...]
    cbt = aug[:C, :]

    mm2 = lax.dot_general(flat, cbt2, (((1,), (0,)), ((), ())),
                          preferred_element_type=jnp.float32)
    x_sqr = jnp.sum(flat * flat, axis=1, keepdims=True)
    cb_sqr = jnp.sum(cbt * cbt, axis=0, keepdims=True)
    dist = (x_sqr + cb_sqr) + mm2

    minval = jnp.min(dist, axis=1, keepdims=True)
    eq = dist == minval
    onehot = jnp.where(eq, 1.0, 0.0)
    agg = lax.dot_general(onehot, aug, (((1,), (1,)), ((), ())),
                          preferred_element_type=jnp.float32)
    codes_ref[0] = agg[:, :C].reshape(32, 32, C)
    idx_f = agg[:, C:C + 1]
    idx_ref[0] = idx_f.astype(jnp.int32).reshape(32, 32)

    nties = agg[:, C + 1:C + 2]
    has_tie = jnp.max(nties) > 1.5

    @pl.when(has_tie)
    def _fix_ties():
        iota_f = lax.broadcasted_iota(jnp.int32, (HW, K), 1).astype(jnp.float32)
        masked = jnp.where(eq, iota_f, jnp.float32(K))
        idx2 = jnp.min(masked, axis=1, keepdims=True)
        onehot2 = jnp.where(masked == idx2, 1.0, 0.0)
        agg2 = lax.dot_general(onehot2, aug, (((1,), (1,)), ((), ())),
                               preferred_element_type=jnp.float32)
        codes_ref[0] = agg2[:, :C].reshape(32, 32, C)
        idx_ref[0] = idx2.astype(jnp.int32).reshape(32, 32)


@jax.jit
def kernel(x, codebook):
    B = x.shape[0]
    xt = jnp.transpose(x, (0, 2, 3, 1))
    cbt = codebook.T
    cbt2 = -2.0 * cbt
    aug = jnp.concatenate(
        [cbt,
         jnp.arange(K, dtype=jnp.float32)[None, :],
         jnp.ones((1, K), jnp.float32)], axis=0)
    codes_bhwc, ind_out = pl.pallas_call(
        _vq_kernel,
        grid=(B,),
        in_specs=[
            pl.BlockSpec((1, 32, 32, C), lambda b: (b, 0, 0, 0)),
            pl.BlockSpec((C, K), lambda b: (0, 0)),
            pl.BlockSpec((C + 2, K), lambda b: (0, 0)),
        ],
        out_specs=[
            pl.BlockSpec((1, 32, 32, C), lambda b: (b, 0, 0, 0)),
            pl.BlockSpec((1, 32, 32), lambda b: (b, 0, 0)),
        ],
        out_shape=[
            jax.ShapeDtypeStruct((B, 32, 32, C), jnp.float32),
            jax.ShapeDtypeStruct((B, 32, 32), jnp.int32),
        ],
    )(xt, cbt2, aug)
    codes_out = jnp.transpose(codes_bhwc, (0, 3, 1, 2))
    return codes_out, ind_out
